# baseline (speedup 1.0000x reference)
import math

import jax
import jax.numpy as jnp
from jax.experimental import pallas as pl
from jax.experimental.pallas import tpu as pltpu

_B = 4
_S = 8
_NP = 512
_D = 3
_H = 64
_LAT = 32
_K = 16
_NL = 3
_RSQRT_H = 1.0 / math.sqrt(float(_H))
_FPS = 8
_EPL = 14
_DPS = 8


def _layernorm(x, s, b, eps=1e-6):
    m = jnp.mean(x, axis=-1, keepdims=True)
    v = jnp.mean((x - m) ** 2, axis=-1, keepdims=True)
    return (x - m) / jnp.sqrt(v + eps) * s + b


def _dotT(a, b):
    return jax.lax.dot_general(
        a, b, dimension_numbers=(((1,), (1,)), ((), ())),
        preferred_element_type=jnp.float32)


def _mm(x, w_ref, b_ref=None):
    y = jnp.dot(x, w_ref[...], preferred_element_type=jnp.float32)
    return y if b_ref is None else y + b_ref[...]


def _encoder_kernel(*refs):
    pts_ref, ptsT_ref, win_ref, bin_ref = refs[:4]
    layer_refs = refs[4:4 + _NL * _EPL]
    out_ref = refs[-1]

    iota_f = jax.lax.broadcasted_iota(
        jnp.int32, (_NP, _NP), 1).astype(jnp.float32)

    for f in range(_FPS):
        pos = pts_ref[f]
        posT = ptsT_ref[f]

        x = _mm(pos, win_ref, bin_ref)

        dist = jnp.zeros((_NP, _NP), jnp.float32)
        for c in range(_D):
            diff = pos[:, c:c + 1] - posT[c:c + 1, :]
            dist = dist + diff * diff

        d = dist
        for _ in range(_K):
            mrow = jnp.min(d, axis=1, keepdims=True)
            cand = jnp.where(d == mrow, iota_f, jnp.float32(_NP))
            amin = jnp.min(cand, axis=1, keepdims=True)
            d = jnp.where(iota_f == amin, jnp.float32(jnp.inf), d)
        mskf = jnp.where(jnp.isinf(d), jnp.float32(1.0), jnp.float32(0.0))

        for l in range(_NL):
            (wq, bq, wk, bk, wv, bv, wpe, bpe, wpd, wat, wout, bout,
             lns, lnb) = layer_refs[l * _EPL:(l + 1) * _EPL]
            q = _mm(x, wq, bq)
            k = _mm(x, wk, bk)
            v = _mm(x, wv, bv)
            pe = _mm(pos, wpe, bpe)
            u = (q + pe) * wat[...][:, 0]
            t = _dotT(u, wpd[...])
            sc = (_dotT(u, k + pe) + _dotT(t, pos)) * _RSQRT_H
            e = jnp.exp(sc) * mskf
            srow = jnp.sum(e, axis=1, keepdims=True)
            out = jnp.dot(e, v, preferred_element_type=jnp.float32) / srow
            out = jax.nn.gelu(_mm(out, wout, bout))
            x = _layernorm(x + out, lns[...], lnb[...])

        out_ref[f] = jnp.max(x, axis=0, keepdims=True)


def _decoder_kernel(*refs):
    (feat_ref, ewq, ebq, ewk, ebk, ewv, ebv, ewo1, ebo1, ewo2, ebo2,
     ln2s, ln2b, wlat, blat, wdec, bdec) = refs[:17]
    step_refs = refs[17:17 + _S * _DPS]
    wc_ref, bc_ref, penc_ref = refs[17 + _S * _DPS:17 + _S * _DPS + 3]
    out_ref = refs[-1]

    penc = penc_ref[...]
    x = feat_ref[...].reshape(_B * _S, _H)
    q = _mm(x, ewq, ebq)
    k = _mm(x, ewk, ebk)
    v = _mm(x, ewv, ebv)
    ri = jax.lax.broadcasted_iota(jnp.int32, (_B * _S, _B * _S), 0)
    ci = jax.lax.broadcasted_iota(jnp.int32, (_B * _S, _B * _S), 1)
    blk = jax.lax.shift_right_logical(ri, 3) == jax.lax.shift_right_logical(ci, 3)
    logits = jnp.where(blk, _dotT(q, k) * _RSQRT_H, jnp.float32(-1e30))
    aw = jax.nn.softmax(logits, axis=1)
    o = jnp.dot(aw, v, preferred_element_type=jnp.float32)
    o = jax.nn.gelu(_mm(o, ewo1, ebo1))
    o = _mm(o, ewo2, ebo2)
    x = _layernorm(x + o, ln2s[...], ln2b[...])
    xm = jnp.concatenate(
        [jnp.max(x[b * _S:(b + 1) * _S], axis=0, keepdims=True)
         for b in range(_B)], axis=0)
    lat = _mm(xm, wlat, blat)
    x0 = jax.nn.gelu(_mm(lat, wdec, bdec))

    rows = [x0]
    for idx in range(_S):
        wdq, bdq, wdk, bdk, wdv, bdv, wdo, bdo = (
            step_refs[idx * _DPS:(idx + 1) * _DPS])
        ln = idx + 1
        xp = jnp.concatenate(rows, axis=0) + jnp.concatenate(
            [jnp.broadcast_to(penc[j:j + 1], (_B, _H)) for j in range(ln)],
            axis=0)
        qd = _mm(rows[ln - 1] + penc[ln - 1:ln], wdq, bdq)
        kd = _mm(xp, wdk, bdk)
        vd = _mm(xp, wdv, bdv)
        rid = jax.lax.broadcasted_iota(jnp.int32, (_B, ln * _B), 0)
        cid = jax.lax.broadcasted_iota(jnp.int32, (_B, ln * _B), 1)
        ok = jax.lax.bitwise_and(cid, _B - 1) == rid
        ld = jnp.where(ok, _dotT(qd, kd) * _RSQRT_H, jnp.float32(-1e30))
        awd = jax.nn.softmax(ld, axis=1)
        od = jnp.dot(awd, vd, preferred_element_type=jnp.float32)
        od = jax.nn.gelu(od)
        od = _mm(od, wdo, bdo)
        rows.append(rows[0] + od)

    for j in range(1, _S + 1):
        c = _mm(rows[j], wc_ref, bc_ref)
        out_ref[:, j - 1:j, :] = c.reshape(_B, 1, -1)


def _fine_kernel(c_ref, wf_ref, bf_ref, flns_ref, flnb_ref, wo_ref, bo_ref,
                 out_ref):
    c = c_ref[...]
    f = _layernorm(_mm(c, wf_ref, bf_ref), flns_ref[...], flnb_ref[...])
    f = jax.nn.gelu(f)
    off = _mm(f, wo_ref, bo_ref)
    out_ref[...] = off + jnp.concatenate([c, c, c, c], axis=1)


def _full_spec(shape):
    return pl.BlockSpec(shape, lambda *_: tuple(0 for _ in shape))


def kernel(points, params):
    nb, ns, npts, d = points.shape
    nf = nb * ns
    pts = points.reshape(nf, npts, d)
    ptsT = pts.transpose(0, 2, 1)

    r2 = lambda a: a.reshape(1, -1)

    enc_ops = [pts, ptsT, params["enc_in"][0], r2(params["enc_in"][1])]
    for p in params["ptl"]:
        enc_ops += [p["q"][0], r2(p["q"][1]), p["k"][0], r2(p["k"][1]),
                    p["v"][0], r2(p["v"][1]), p["pe"][0], r2(p["pe"][1]),
                    p["pd"][0], p["attn"][0], p["out"][0], r2(p["out"][1]),
                    r2(p["ln_s"]), r2(p["ln_b"])]

    in_specs = [pl.BlockSpec((_FPS, npts, d), lambda i: (i, 0, 0)),
                pl.BlockSpec((_FPS, d, npts), lambda i: (i, 0, 0))]
    in_specs += [_full_spec(o.shape) for o in enc_ops[2:]]

    feat = pl.pallas_call(
        _encoder_kernel,
        grid=(nf // _FPS,),
        in_specs=in_specs,
        out_specs=pl.BlockSpec((_FPS, 1, _H), lambda i: (i, 0, 0)),
        out_shape=jax.ShapeDtypeStruct((nf, 1, _H), jnp.float32),
        compiler_params=pltpu.CompilerParams(
            dimension_semantics=("parallel",),
            vmem_limit_bytes=100 * 1024 * 1024),
    )(*enc_ops)

    feat = feat.reshape(nb, ns, _H)

    position = jnp.arange(ns + 1, dtype=jnp.float32)[:, None]
    div_term = jnp.exp(jnp.arange(0, _H, 2, dtype=jnp.float32)
                       * (-math.log(10000.0) / _H))
    penc = jnp.stack([jnp.sin(position * div_term),
                      jnp.cos(position * div_term)], axis=-1).reshape(ns + 1, _H)

    e = params["etl"]
    dec_ops = [feat,
               e["q"][0], r2(e["q"][1]), e["k"][0], r2(e["k"][1]),
               e["v"][0], r2(e["v"][1]), e["o1"][0], r2(e["o1"][1]),
               e["o2"][0], r2(e["o2"][1]),
               r2(params["ln2_s"]), r2(params["ln2_b"]),
               params["latent"][0], r2(params["latent"][1]),
               params["dec_in"][0], r2(params["dec_in"][1])]
    for i in range(ns):
        dec_ops += [params["dq"][i][0], r2(params["dq"][i][1]),
                    params["dk"][i][0], r2(params["dk"][i][1]),
                    params["dv"][i][0], r2(params["dv"][i][1]),
                    params["do"][i][0], r2(params["do"][i][1])]
    dec_ops += [params["coarse"][0], r2(params["coarse"][1]), penc]

    coarse = pl.pallas_call(
        _decoder_kernel,
        in_specs=[_full_spec(o.shape) for o in dec_ops],
        out_specs=_full_spec((nb, ns, npts // 4 * d)),
        out_shape=jax.ShapeDtypeStruct((nb, ns, npts // 4 * d), jnp.float32),
    )(*dec_ops)

    coarse_flat = coarse.reshape(nb * ns * (npts // 4), d)

    fine_ops = [coarse_flat, params["feat"][0], r2(params["feat"][1]),
                r2(params["fln_s"]), r2(params["fln_b"]),
                params["off"][0], r2(params["off"][1])]

    fine_flat = pl.pallas_call(
        _fine_kernel,
        in_specs=[_full_spec(o.shape) for o in fine_ops],
        out_specs=_full_spec((nb * ns * (npts // 4), 4 * d)),
        out_shape=jax.ShapeDtypeStruct((nb * ns * (npts // 4), 4 * d),
                                       jnp.float32),
    )(*fine_ops)

    return fine_flat.reshape(nb, ns, npts, d)


# speedup vs baseline: 1.2595x; 1.2595x over previous
import math

import jax
import jax.numpy as jnp
from jax.experimental import pallas as pl
from jax.experimental.pallas import tpu as pltpu

_B = 4
_S = 8
_NP = 512
_D = 3
_H = 64
_LAT = 32
_K = 16
_NL = 3
_RSQRT_H = 1.0 / math.sqrt(float(_H))
_FPS = 4
_EPL = 14
_DPS = 8


def _layernorm(x, s, b, eps=1e-6):
    m = jnp.mean(x, axis=-1, keepdims=True)
    v = jnp.mean((x - m) ** 2, axis=-1, keepdims=True)
    return (x - m) / jnp.sqrt(v + eps) * s + b


def _dotT(a, b):
    return jax.lax.dot_general(
        a, b, dimension_numbers=(((1,), (1,)), ((), ())),
        preferred_element_type=jnp.float32)


def _mm(x, w_ref, b_ref=None):
    y = jnp.dot(x, w_ref[...], preferred_element_type=jnp.float32)
    return y if b_ref is None else y + b_ref[...]


def _encoder_kernel(*refs):
    pts_ref, ptsT_ref, win_ref, bin_ref = refs[:4]
    layer_refs = refs[4:4 + _NL * _EPL]
    out_ref = refs[-1]

    iota_f = jax.lax.broadcasted_iota(
        jnp.int32, (_NP, _NP), 1).astype(jnp.float32)

    for f in range(_FPS):
        pos = pts_ref[f]
        posT = ptsT_ref[f]

        x = _mm(pos, win_ref, bin_ref)

        dist = jnp.zeros((_NP, _NP), jnp.float32)
        for c in range(_D):
            diff = pos[:, c:c + 1] - posT[c:c + 1, :]
            dist = dist + diff * diff

        d = dist
        for _ in range(_K):
            mrow = jnp.min(d, axis=1, keepdims=True)
            cand = jnp.where(d == mrow, iota_f, jnp.float32(_NP))
            amin = jnp.min(cand, axis=1, keepdims=True)
            d = jnp.where(iota_f == amin, jnp.float32(jnp.inf), d)
        mskf = jnp.where(jnp.isinf(d), jnp.float32(1.0), jnp.float32(0.0))

        for l in range(_NL):
            (wq, bq, wk, bk, wv, bv, wpe, bpe, wpd, wat, wout, bout,
             lns, lnb) = layer_refs[l * _EPL:(l + 1) * _EPL]
            q = _mm(x, wq, bq)
            k = _mm(x, wk, bk)
            v = _mm(x, wv, bv)
            pe = _mm(pos, wpe, bpe)
            u = (q + pe) * wat[...][:, 0]
            t = _dotT(u, wpd[...])
            sc = (_dotT(u, k + pe) + _dotT(t, pos)) * _RSQRT_H
            e = jnp.exp(sc) * mskf
            srow = jnp.sum(e, axis=1, keepdims=True)
            out = jnp.dot(e, v, preferred_element_type=jnp.float32) / srow
            out = jax.nn.gelu(_mm(out, wout, bout))
            x = _layernorm(x + out, lns[...], lnb[...])

        out_ref[f] = jnp.max(x, axis=0, keepdims=True)


def _decoder_kernel(*refs):
    (feat_ref, ewq, ebq, ewk, ebk, ewv, ebv, ewo1, ebo1, ewo2, ebo2,
     ln2s, ln2b, wlat, blat, wdec, bdec) = refs[:17]
    step_refs = refs[17:17 + _S * _DPS]
    wc_ref, bc_ref, penc_ref = refs[17 + _S * _DPS:17 + _S * _DPS + 3]
    out_ref = refs[-1]

    penc = penc_ref[...]
    x = feat_ref[...].reshape(_B * _S, _H)
    q = _mm(x, ewq, ebq)
    k = _mm(x, ewk, ebk)
    v = _mm(x, ewv, ebv)
    ri = jax.lax.broadcasted_iota(jnp.int32, (_B * _S, _B * _S), 0)
    ci = jax.lax.broadcasted_iota(jnp.int32, (_B * _S, _B * _S), 1)
    blk = jax.lax.shift_right_logical(ri, 3) == jax.lax.shift_right_logical(ci, 3)
    logits = jnp.where(blk, _dotT(q, k) * _RSQRT_H, jnp.float32(-1e30))
    aw = jax.nn.softmax(logits, axis=1)
    o = jnp.dot(aw, v, preferred_element_type=jnp.float32)
    o = jax.nn.gelu(_mm(o, ewo1, ebo1))
    o = _mm(o, ewo2, ebo2)
    x = _layernorm(x + o, ln2s[...], ln2b[...])
    xm = jnp.concatenate(
        [jnp.max(x[b * _S:(b + 1) * _S], axis=0, keepdims=True)
         for b in range(_B)], axis=0)
    lat = _mm(xm, wlat, blat)
    x0 = jax.nn.gelu(_mm(lat, wdec, bdec))

    rows = [x0]
    for idx in range(_S):
        wdq, bdq, wdk, bdk, wdv, bdv, wdo, bdo = (
            step_refs[idx * _DPS:(idx + 1) * _DPS])
        ln = idx + 1
        xp = jnp.concatenate(rows, axis=0) + jnp.concatenate(
            [jnp.broadcast_to(penc[j:j + 1], (_B, _H)) for j in range(ln)],
            axis=0)
        qd = _mm(rows[ln - 1] + penc[ln - 1:ln], wdq, bdq)
        kd = _mm(xp, wdk, bdk)
        vd = _mm(xp, wdv, bdv)
        rid = jax.lax.broadcasted_iota(jnp.int32, (_B, ln * _B), 0)
        cid = jax.lax.broadcasted_iota(jnp.int32, (_B, ln * _B), 1)
        ok = jax.lax.bitwise_and(cid, _B - 1) == rid
        ld = jnp.where(ok, _dotT(qd, kd) * _RSQRT_H, jnp.float32(-1e30))
        awd = jax.nn.softmax(ld, axis=1)
        od = jnp.dot(awd, vd, preferred_element_type=jnp.float32)
        od = jax.nn.gelu(od)
        od = _mm(od, wdo, bdo)
        rows.append(rows[0] + od)

    for j in range(1, _S + 1):
        c = _mm(rows[j], wc_ref, bc_ref)
        out_ref[:, j - 1:j, :] = c.reshape(_B, 1, -1)


def _fine_kernel(c_ref, wf_ref, bf_ref, flns_ref, flnb_ref, wo_ref, bo_ref,
                 out_ref):
    c = c_ref[...]
    f = _layernorm(_mm(c, wf_ref, bf_ref), flns_ref[...], flnb_ref[...])
    f = jax.nn.gelu(f)
    off = _mm(f, wo_ref, bo_ref)
    out_ref[...] = off + jnp.concatenate([c, c, c, c], axis=1)


def _full_spec(shape):
    return pl.BlockSpec(shape, lambda *_: tuple(0 for _ in shape))


def kernel(points, params):
    nb, ns, npts, d = points.shape
    nf = nb * ns
    pts = points.reshape(nf, npts, d)
    ptsT = pts.transpose(0, 2, 1)

    r2 = lambda a: a.reshape(1, -1)

    enc_ops = [pts, ptsT, params["enc_in"][0], r2(params["enc_in"][1])]
    for p in params["ptl"]:
        enc_ops += [p["q"][0], r2(p["q"][1]), p["k"][0], r2(p["k"][1]),
                    p["v"][0], r2(p["v"][1]), p["pe"][0], r2(p["pe"][1]),
                    p["pd"][0], p["attn"][0], p["out"][0], r2(p["out"][1]),
                    r2(p["ln_s"]), r2(p["ln_b"])]

    in_specs = [pl.BlockSpec((_FPS, npts, d), lambda i: (i, 0, 0)),
                pl.BlockSpec((_FPS, d, npts), lambda i: (i, 0, 0))]
    in_specs += [_full_spec(o.shape) for o in enc_ops[2:]]

    feat = pl.pallas_call(
        _encoder_kernel,
        grid=(nf // _FPS,),
        in_specs=in_specs,
        out_specs=pl.BlockSpec((_FPS, 1, _H), lambda i: (i, 0, 0)),
        out_shape=jax.ShapeDtypeStruct((nf, 1, _H), jnp.float32),
        compiler_params=pltpu.CompilerParams(
            dimension_semantics=("parallel",)),
    )(*enc_ops)

    feat = feat.reshape(nb, ns, _H)

    position = jnp.arange(ns + 1, dtype=jnp.float32)[:, None]
    div_term = jnp.exp(jnp.arange(0, _H, 2, dtype=jnp.float32)
                       * (-math.log(10000.0) / _H))
    penc = jnp.stack([jnp.sin(position * div_term),
                      jnp.cos(position * div_term)], axis=-1).reshape(ns + 1, _H)

    e = params["etl"]
    dec_ops = [feat,
               e["q"][0], r2(e["q"][1]), e["k"][0], r2(e["k"][1]),
               e["v"][0], r2(e["v"][1]), e["o1"][0], r2(e["o1"][1]),
               e["o2"][0], r2(e["o2"][1]),
               r2(params["ln2_s"]), r2(params["ln2_b"]),
               params["latent"][0], r2(params["latent"][1]),
               params["dec_in"][0], r2(params["dec_in"][1])]
    for i in range(ns):
        dec_ops += [params["dq"][i][0], r2(params["dq"][i][1]),
                    params["dk"][i][0], r2(params["dk"][i][1]),
                    params["dv"][i][0], r2(params["dv"][i][1]),
                    params["do"][i][0], r2(params["do"][i][1])]
    dec_ops += [params["coarse"][0], r2(params["coarse"][1]), penc]

    coarse = pl.pallas_call(
        _decoder_kernel,
        in_specs=[_full_spec(o.shape) for o in dec_ops],
        out_specs=_full_spec((nb, ns, npts // 4 * d)),
        out_shape=jax.ShapeDtypeStruct((nb, ns, npts // 4 * d), jnp.float32),
    )(*dec_ops)

    coarse_flat = coarse.reshape(nb * ns * (npts // 4), d)

    fine_ops = [coarse_flat, params["feat"][0], r2(params["feat"][1]),
                r2(params["fln_s"]), r2(params["fln_b"]),
                params["off"][0], r2(params["off"][1])]

    fine_flat = pl.pallas_call(
        _fine_kernel,
        in_specs=[_full_spec(o.shape) for o in fine_ops],
        out_specs=_full_spec((nb * ns * (npts // 4), 4 * d)),
        out_shape=jax.ShapeDtypeStruct((nb * ns * (npts // 4), 4 * d),
                                       jnp.float32),
    )(*fine_ops)

    return fine_flat.reshape(nb, ns, npts, d)
